# Initial kernel scaffold; baseline (speedup 1.0000x reference)
import jax
import jax.numpy as jnp
from jax.experimental import pallas as pl


def kernel(x):
    raise NotImplementedError("write your pallas kernel here")


# speedup vs baseline: 6.6961x; 6.6961x over previous
import functools

import jax
import jax.numpy as jnp
from jax import lax
from jax.experimental import pallas as pl
from jax.experimental.pallas import tpu as pltpu

_EPS = 1e-6
_TINY = 1.1754943508222875e-38
_KS0 = 0
_KS1 = 42
_KS2 = 0x1BD11BF0
_ROT = ((13, 15, 26, 6), (17, 29, 16, 24))


def _threefry_bits(n):
    ks = (_KS0, _KS1, _KS2)
    x0 = jnp.zeros_like(n)
    x1 = n + jnp.uint32(_KS1)
    for i in range(5):
        for r in _ROT[i % 2]:
            x0 = x0 + x1
            x1 = (x1 << r) | (x1 >> (32 - r))
            x1 = x1 ^ x0
        x0 = x0 + jnp.uint32(ks[(i + 1) % 3])
        x1 = x1 + jnp.uint32(ks[(i + 2) % 3] + (i + 1))
    return x0 ^ x1


def _gumbel(n):
    bits = _threefry_bits(n)
    mant = (bits >> 9) | jnp.uint32(0x3F800000)
    u0 = pltpu.bitcast(mant, jnp.float32) - 1.0
    u = jnp.maximum(_TINY, u0 + _TINY)
    return -jnp.log(-jnp.log(u))


def _shift_m1(a, axis):
    n = a.shape[axis]
    return jnp.concatenate(
        [lax.slice_in_dim(a, 1, n, axis=axis),
         lax.slice_in_dim(a, 0, 1, axis=axis)], axis=axis)


def _shift_p1(a, axis):
    n = a.shape[axis]
    return jnp.concatenate(
        [lax.slice_in_dim(a, n - 1, n, axis=axis),
         lax.slice_in_dim(a, 0, n - 1, axis=axis)], axis=axis)


def _pool_kernel(x_ref, o_ref, *, Hout, Wout, L):
    xv = x_ref[0]
    H, W = xv.shape
    b = pl.program_id(0)

    x01 = _shift_m1(xv, 1)
    x10 = _shift_m1(xv, 0)
    x11 = _shift_m1(x10, 1)
    fs = (xv, x01, x10, x11)
    denom = ((xv + x01) + x10) + x11 + _EPS

    hh = lax.broadcasted_iota(jnp.int32, (H, W), 0)
    ww = lax.broadcasted_iota(jnp.int32, (H, W), 1)
    l = hh * Wout + ww
    base = b * (4 * L) + l

    best = None
    idx = None
    for q in range(4):
        g = _gumbel((base + q * L).astype(jnp.uint32))
        v = jnp.log(jnp.maximum(fs[q] / denom, 1e-30)) + g
        if q == 0:
            best, idx = v, jnp.zeros_like(l)
        else:
            take = v > best
            idx = jnp.where(take, q, idx)
            best = jnp.maximum(best, v)

    valid = (hh < Hout) & (ww < Wout)
    c0 = ((idx == 0) & valid).astype(jnp.float32)
    c1 = ((idx == 1) & valid).astype(jnp.float32)
    c2 = ((idx == 2) & valid).astype(jnp.float32)
    c3 = ((idx == 3) & valid).astype(jnp.float32)
    m = c0 + _shift_p1(c1, 1) + _shift_p1(c2 + _shift_p1(c3, 1), 0)

    inv_r = jnp.where((hh == 0) | (hh == H - 1), 1.0, 0.5)
    inv_c = jnp.where((ww == 0) | (ww == W - 1), 1.0, 0.5)
    o_ref[0] = (xv * m) * (inv_r * inv_c)


@jax.jit
def kernel(x):
    B, C, H, W = x.shape
    bc = B * C
    Hout, Wout = H - 1, W - 1
    L = Hout * Wout
    xr = x.reshape(bc, H, W)
    body = functools.partial(_pool_kernel, Hout=Hout, Wout=Wout, L=L)
    out = pl.pallas_call(
        body,
        grid=(bc,),
        in_specs=[pl.BlockSpec((1, H, W), lambda b: (b, 0, 0))],
        out_specs=pl.BlockSpec((1, H, W), lambda b: (b, 0, 0)),
        out_shape=jax.ShapeDtypeStruct((bc, H, W), x.dtype),
        compiler_params=pltpu.CompilerParams(
            dimension_semantics=("parallel",)),
    )(xr)
    return out.reshape(B, C, H, W)
